# baseline (speedup 1.0000x reference)
import functools
import math

import jax
import jax.numpy as jnp
from jax import lax
from jax.experimental import pallas as pl
from jax.experimental.pallas import tpu as pltpu
from jax.experimental.pallas import tpu_sc as plsc

D_EMBED = 64
SCALE = math.sqrt(D_EMBED)
L = 16
C = 128
NBUF = 4


def _build_sc_kernel(num_rows_x, num_cols_x):
    try:
        info = plsc.get_sparse_core_info()
        nc, ns = info.num_cores, info.num_subcores
    except Exception:
        nc, ns = 2, 16
    nw = nc * ns
    b_total = num_rows_x * num_cols_x
    assert num_rows_x % C == 0
    npairs = b_total // C
    assert npairs % nw == 0
    per_w = npairs // nw
    assert per_w % NBUF == 0 and per_w >= 2 * NBUF
    blocks_per_col = num_rows_x // C

    mesh = plsc.VectorSubcoreMesh(core_axis_name="c", subcore_axis_name="s")

    @functools.partial(
        pl.kernel,
        mesh=mesh,
        compiler_params=pltpu.CompilerParams(
            use_tc_tiling_on_sc=False, needs_layout_passes=False),
        out_type=jax.ShapeDtypeStruct(
            (num_cols_x, D_EMBED, num_rows_x), jnp.float32),
        scratch_types=(
            [pltpu.VMEM((per_w, C), jnp.int32)]
            + [pltpu.VMEM((C, D_EMBED), jnp.float32) for _ in range(NBUF)]
            + [pltpu.VMEM((D_EMBED, C), jnp.float32) for _ in range(NBUF)]
            + [pltpu.SemaphoreType.DMA for _ in range(2 * NBUF)]
        ),
    )
    def emb(x_hbm, table_hbm, out_hbm, idx_v, *bufs_and_sems):
        gbuf = bufs_and_sems[0:NBUF]
        tbuf = bufs_and_sems[NBUF:2 * NBUF]
        gsem = bufs_and_sems[2 * NBUF:3 * NBUF]
        ssem = bufs_and_sems[3 * NBUF:4 * NBUF]

        wid = lax.axis_index("s") * nc + lax.axis_index("c")
        base_p = wid * per_w

        pltpu.sync_copy(x_hbm.at[wid], idx_v)

        def start_gather(pl_local, b):
            pltpu.async_copy(
                table_hbm.at[idx_v.at[pl_local]], gbuf[b], gsem[b])

        def wait_gather(b):
            pltpu.make_async_copy(
                table_hbm.at[idx_v.at[0]], gbuf[b], gsem[b]).wait()

        def start_write(p_local, b):
            p = base_p + p_local
            j = p // blocks_per_col
            ci = p % blocks_per_col
            pltpu.async_copy(
                tbuf[b], out_hbm.at[j, :, pl.ds(ci * C, C)], ssem[b])

        def wait_write(b):
            pltpu.make_async_copy(
                tbuf[b], out_hbm.at[0, :, pl.ds(0, C)], ssem[b]).wait()

        lane_iota = jax.lax.iota(jnp.int32, L)

        def transpose_scale(b):
            gb, tb = gbuf[b], tbuf[b]

            @plsc.parallel_loop(0, C, unroll=2)
            def _(r):
                col = jnp.full((L,), r, jnp.int32)
                for c4 in range(D_EMBED // L):
                    v = gb[r, pl.ds(c4 * L, L)] * SCALE
                    plsc.store_scatter(tb, [lane_iota + (c4 * L), col], v)

        for b in range(NBUF):
            start_gather(b, b)

        for b in range(NBUF):
            wait_gather(b)
            transpose_scale(b)
            start_write(b, b)
            start_gather(b + NBUF, b)

        @pl.loop(NBUF, per_w - NBUF, step=NBUF)
        def _(g):
            for b in range(NBUF):
                p = g + b
                wait_gather(b)
                wait_write(b)
                transpose_scale(b)
                start_write(p, b)
                start_gather(p + NBUF, b)

        for b in range(NBUF):
            p = per_w - NBUF + b
            wait_gather(b)
            wait_write(b)
            transpose_scale(b)
            start_write(p, b)

        for b in range(NBUF):
            wait_write(b)

    return emb, nw, per_w


def kernel(x, table):
    rows, cols = x.shape
    emb, nw, per_w = _build_sc_kernel(rows, cols)
    xw = x.T.reshape(nw, per_w, C).astype(jnp.int32)
    out_t = emb(xw, table)
    return out_t.transpose(2, 0, 1)


# speedup vs baseline: 1.2081x; 1.2081x over previous
import functools
import math

import jax
import jax.numpy as jnp
from jax import lax
from jax.experimental import pallas as pl
from jax.experimental.pallas import tpu as pltpu
from jax.experimental.pallas import tpu_sc as plsc

D_EMBED = 64
SCALE = math.sqrt(D_EMBED)
L = 16
C = 128
NBUF = 4


def _magic_div(q, d):
    shift = 21
    magic = -(-(1 << shift) // d)
    r = jax.lax.shift_right_logical(q * magic, shift)
    return r


def _build_sc_kernel(num_rows_x, num_cols_x):
    try:
        info = plsc.get_sparse_core_info()
        nc, ns = info.num_cores, info.num_subcores
    except Exception:
        nc, ns = 2, 16
    nw = nc * ns
    b_total = num_rows_x * num_cols_x
    assert b_total % (nw * C) == 0
    per_w = b_total // nw
    nchunk = per_w // C
    rows_w = num_rows_x // nw
    assert rows_w * num_cols_x == per_w
    assert nchunk % NBUF == 0 and nchunk >= 2 * NBUF

    mesh = plsc.VectorSubcoreMesh(core_axis_name="c", subcore_axis_name="s")

    @functools.partial(
        pl.kernel,
        mesh=mesh,
        compiler_params=pltpu.CompilerParams(
            use_tc_tiling_on_sc=False, needs_layout_passes=False),
        out_type=jax.ShapeDtypeStruct((b_total, D_EMBED), jnp.float32),
        scratch_types=(
            [pltpu.VMEM((rows_w, num_cols_x), jnp.int32),
             pltpu.VMEM((nchunk, C), jnp.int32)]
            + [pltpu.VMEM((C, D_EMBED), jnp.float32) for _ in range(2 * NBUF)]
            + [pltpu.SemaphoreType.DMA for _ in range(2 * NBUF)]
        ),
    )
    def emb(x_hbm, table_hbm, out_hbm, xstage, idx_v, *bufs_and_sems):
        gbuf = bufs_and_sems[0:NBUF]
        sbuf = bufs_and_sems[NBUF:2 * NBUF]
        gsem = bufs_and_sems[2 * NBUF:3 * NBUF]
        ssem = bufs_and_sems[3 * NBUF:4 * NBUF]

        wid = lax.axis_index("s") * nc + lax.axis_index("c")
        base = wid * per_w

        pltpu.sync_copy(x_hbm.at[pl.ds(wid * rows_w, rows_w)], xstage)

        lane_iota = jax.lax.iota(jnp.int32, L)

        @plsc.parallel_loop(0, per_w // L, unroll=4)
        def _(t):
            q = lane_iota + t * L
            r = _magic_div(q, num_cols_x)
            c = q - r * num_cols_x
            v = plsc.load_gather(xstage, [r, c])
            cchunk = t // (C // L)
            coff = t % (C // L)
            idx_v[cchunk, pl.ds(coff * L, L)] = v

        def start_gather(j, b):
            pltpu.async_copy(table_hbm.at[idx_v.at[j]], gbuf[b], gsem[b])

        def wait_gather(b):
            pltpu.make_async_copy(
                table_hbm.at[idx_v.at[0]], gbuf[b], gsem[b]).wait()

        def start_scatter(j, b):
            pltpu.async_copy(
                sbuf[b], out_hbm.at[pl.ds(base + j * C, C)], ssem[b])

        def wait_scatter(b):
            pltpu.make_async_copy(
                sbuf[b], out_hbm.at[pl.ds(base, C)], ssem[b]).wait()

        def scale(b):
            gb, sb = gbuf[b], sbuf[b]

            @plsc.parallel_loop(0, C, unroll=2)
            def _(r):
                for c4 in range(D_EMBED // L):
                    sl = pl.ds(c4 * L, L)
                    sb[r, sl] = gb[r, sl] * SCALE

        for b in range(NBUF):
            start_gather(b, b)

        for b in range(NBUF):
            wait_gather(b)
            scale(b)
            start_scatter(b, b)
            start_gather(b + NBUF, b)

        @pl.loop(NBUF, nchunk - NBUF, step=NBUF)
        def _(g):
            for b in range(NBUF):
                j = g + b
                wait_gather(b)
                wait_scatter(b)
                scale(b)
                start_scatter(j, b)
                start_gather(j + NBUF, b)

        for b in range(NBUF):
            j = nchunk - NBUF + b
            wait_gather(b)
            wait_scatter(b)
            scale(b)
            start_scatter(j, b)

        for b in range(NBUF):
            wait_scatter(b)

    return emb


def kernel(x, table):
    rows, cols = x.shape
    emb = _build_sc_kernel(rows, cols)
    out = emb(x, table)
    return out.reshape(rows, cols, D_EMBED)
